# Initial kernel scaffold; baseline (speedup 1.0000x reference)
import functools

import jax
import jax.numpy as jnp
import numpy as np
from jax import lax
from jax.experimental import pallas as pl
from jax.experimental.pallas import tpu as pltpu
from jax.experimental.pallas import tpu_sc as plsc

N_NODES = 10000
N_EDGES = 320000
D = 128

NC = 2
NS = 16
NW = NC * NS

CH = 128
N_CHUNKS = N_EDGES // CH
CPT = -(-N_CHUNKS // NW)
ACC_N = 10016
RPT = ACC_N // NS
PAD_ROW = N_NODES

_w = np.arange(NW)
_CNT = (N_CHUNKS // NW) + (_w < (N_CHUNKS % NW)).astype(np.int64)
_START = np.concatenate([[0], np.cumsum(_CNT)[:-1]])
_j = np.arange(CPT)
_SLOT_CHUNK = (_START[:, None]
               + np.minimum(_j[None, :], _CNT[:, None] - 1)).astype(np.int32)
_SLOT_VALID = (_j[None, :] < _CNT[:, None])


def _sc_segment_sum(edge_features, idx_slots):
    mesh = plsc.VectorSubcoreMesh(core_axis_name="c", subcore_axis_name="s")

    @functools.partial(
        pl.kernel,
        out_type=jax.ShapeDtypeStruct((NC, ACC_N, D), jnp.float32),
        mesh=mesh,
        scratch_types=[
            pltpu.VMEM((CPT, CH), jnp.int32),
            pltpu.VMEM((CH, D), jnp.float32),
            pltpu.VMEM_SHARED((ACC_N, D), jnp.float32),
        ],
    )
    def body(edge_hbm, idx_hbm, out_hbm, idx_v, ebuf, acc):
        c = lax.axis_index("c")
        s = lax.axis_index("s")
        wid = c * NS + s

        def zrow(r, _):
            for q in range(D // 16):
                ebuf[r, pl.ds(q * 16, 16)] = jnp.zeros((16,), jnp.float32)
            return 0
        lax.fori_loop(0, CH, zrow, 0)
        base_r = s * RPT
        nfull = RPT // CH
        for k in range(nfull):
            pltpu.sync_copy(ebuf, acc.at[pl.ds(base_r + k * CH, CH)])
        rem = RPT - nfull * CH
        if rem:
            pltpu.sync_copy(ebuf.at[pl.ds(0, rem)],
                            acc.at[pl.ds(base_r + nfull * CH, rem)])
        plsc.subcore_barrier()

        pltpu.sync_copy(idx_hbm.at[wid], idx_v)

        nb = jnp.int32(N_CHUNKS // NW)
        rm = jnp.int32(N_CHUNKS % NW)
        cnt = nb + jnp.where(wid < rm, jnp.int32(1), jnp.int32(0))
        start = wid * nb + jnp.minimum(wid, rm)

        def chunk_body(j, _):
            ch = start + jnp.minimum(j, cnt - 1)
            pltpu.sync_copy(edge_hbm.at[pl.ds(ch * CH, CH)], ebuf)
            pltpu.sync_copy(ebuf, acc.at[idx_v.at[j]], add=True)
            return 0
        lax.fori_loop(0, CPT, chunk_body, 0)

        plsc.subcore_barrier()
        pltpu.sync_copy(acc.at[pl.ds(base_r, RPT)],
                        out_hbm.at[c, pl.ds(base_r, RPT)])

    return body(edge_features, idx_slots)


def _tc_mlp(partials, node_features, W1a, W1b, b1, W2, b2, W3, b3, gamma, beta):
    BN = 1000
    grid = N_NODES // BN

    def body(p_ref, nf_ref, w1a_ref, w1b_ref, b1_ref, w2_ref, b2_ref,
             w3_ref, b3_ref, g_ref, bt_ref, out_ref):
        hd = p_ref[0] + p_ref[1]
        nf = nf_ref[...]
        h = (jnp.dot(hd, w1a_ref[...], preferred_element_type=jnp.float32)
             + jnp.dot(nf, w1b_ref[...], preferred_element_type=jnp.float32)
             + b1_ref[...])
        h = jnp.maximum(h, 0.0)
        h = jnp.maximum(
            jnp.dot(h, w2_ref[...], preferred_element_type=jnp.float32)
            + b2_ref[...], 0.0)
        h = (jnp.dot(h, w3_ref[...], preferred_element_type=jnp.float32)
             + b3_ref[...])
        mean = jnp.mean(h, axis=-1, keepdims=True)
        cent = h - mean
        var = jnp.mean(cent * cent, axis=-1, keepdims=True)
        h = cent * lax.rsqrt(var + 1e-5) * g_ref[...] + bt_ref[...]
        out_ref[...] = h + nf

    full = lambda shape: pl.BlockSpec(shape, lambda i: (0,) * len(shape))
    return pl.pallas_call(
        body,
        grid=(grid,),
        in_specs=[
            pl.BlockSpec((NC, BN, D), lambda i: (0, i, 0)),
            pl.BlockSpec((BN, D), lambda i: (i, 0)),
            full((D, D)), full((D, D)), full((1, D)),
            full((D, D)), full((1, D)),
            full((D, D)), full((1, D)),
            full((1, D)), full((1, D)),
        ],
        out_specs=pl.BlockSpec((BN, D), lambda i: (i, 0)),
        out_shape=jax.ShapeDtypeStruct((N_NODES, D), jnp.float32),
    )(partials, node_features, W1a, W1b, b1, W2, b2, W3, b3, gamma, beta)


def kernel(node_features, edge_features, edge_index, W1, b1, W2, b2, W3, b3,
           gamma, beta):
    dst = edge_index[1].astype(jnp.int32)
    chunks = dst.reshape(N_CHUNKS, CH)
    idx_slots = jnp.where(jnp.asarray(_SLOT_VALID)[:, :, None],
                          chunks[jnp.asarray(_SLOT_CHUNK)],
                          jnp.int32(PAD_ROW))

    partials = _sc_segment_sum(edge_features, idx_slots)

    W1a = W1[:D]
    W1b = W1[D:]
    r1 = lambda v: v.reshape(1, D)
    return _tc_mlp(partials, node_features, W1a, W1b, r1(b1), W2, r1(b2),
                   W3, r1(b3), r1(gamma), r1(beta))


# speedup vs baseline: 4.9379x; 4.9379x over previous
import functools

import jax
import jax.numpy as jnp
import numpy as np
from jax import lax
from jax.experimental import pallas as pl
from jax.experimental.pallas import tpu as pltpu
from jax.experimental.pallas import tpu_sc as plsc

N_NODES = 10000
N_EDGES = 320000
D = 128

NC = 2
NS = 16
NW = NC * NS

CH = 128
N_CHUNKS = N_EDGES // CH
CPT = -(-N_CHUNKS // NW)
ACC_N = 10112
RPT = ACC_N // NS
PAD_ROW = N_NODES

_w = np.arange(NW)
_CNT = (N_CHUNKS // NW) + (_w < (N_CHUNKS % NW)).astype(np.int64)
_START = np.concatenate([[0], np.cumsum(_CNT)[:-1]])
_j = np.arange(CPT)
_SLOT_CHUNK = (_START[:, None]
               + np.minimum(_j[None, :], _CNT[:, None] - 1)).astype(np.int32)
_SLOT_VALID = (_j[None, :] < _CNT[:, None])


def _sc_segment_sum(edge_features, idx_slots):
    mesh = plsc.VectorSubcoreMesh(core_axis_name="c", subcore_axis_name="s")

    @functools.partial(
        pl.kernel,
        out_type=jax.ShapeDtypeStruct((NC, ACC_N, D), jnp.float32),
        mesh=mesh,
        scratch_types=[
            pltpu.VMEM((CPT, CH), jnp.int32),
            pltpu.VMEM((CH, D), jnp.float32),
            pltpu.VMEM_SHARED((ACC_N, D), jnp.float32),
        ],
    )
    def body(edge_hbm, idx_hbm, out_hbm, idx_v, ebuf, acc):
        c = lax.axis_index("c")
        s = lax.axis_index("s")
        wid = c * NS + s

        def zrow(r, _):
            for q in range(D // 16):
                ebuf[r, pl.ds(q * 16, 16)] = jnp.zeros((16,), jnp.float32)
            return 0
        lax.fori_loop(0, CH, zrow, 0)
        base_r = s * RPT
        nfull = RPT // CH
        for k in range(nfull):
            pltpu.sync_copy(ebuf, acc.at[pl.ds(base_r + k * CH, CH)])
        rem = RPT - nfull * CH
        if rem:
            pltpu.sync_copy(ebuf.at[pl.ds(0, rem)],
                            acc.at[pl.ds(base_r + nfull * CH, rem)])
        plsc.subcore_barrier()

        pltpu.sync_copy(idx_hbm.at[wid], idx_v)

        nb = jnp.int32(N_CHUNKS // NW)
        rm = jnp.int32(N_CHUNKS % NW)
        cnt = nb + jnp.where(wid < rm, jnp.int32(1), jnp.int32(0))
        start = wid * nb + jnp.minimum(wid, rm)

        def chunk_body(j, _):
            ch = start + jnp.minimum(j, cnt - 1)
            pltpu.sync_copy(edge_hbm.at[pl.ds(ch * CH, CH)], ebuf)
            pltpu.sync_copy(ebuf, acc.at[idx_v.at[j]], add=True)
            return 0
        lax.fori_loop(0, CPT, chunk_body, 0)

        plsc.subcore_barrier()
        pltpu.sync_copy(acc.at[pl.ds(base_r, RPT)],
                        out_hbm.at[c, pl.ds(base_r, RPT)])

    return body(edge_features, idx_slots)


def _tc_mlp(partials, node_features, W1a, W1b, b1, W2, b2, W3, b3, gamma, beta):
    BN = 1000
    grid = N_NODES // BN

    def body(p_ref, nf_ref, w1a_ref, w1b_ref, b1_ref, w2_ref, b2_ref,
             w3_ref, b3_ref, g_ref, bt_ref, out_ref):
        hd = p_ref[0] + p_ref[1]
        nf = nf_ref[...]
        h = (jnp.dot(hd, w1a_ref[...], preferred_element_type=jnp.float32)
             + jnp.dot(nf, w1b_ref[...], preferred_element_type=jnp.float32)
             + b1_ref[...])
        h = jnp.maximum(h, 0.0)
        h = jnp.maximum(
            jnp.dot(h, w2_ref[...], preferred_element_type=jnp.float32)
            + b2_ref[...], 0.0)
        h = (jnp.dot(h, w3_ref[...], preferred_element_type=jnp.float32)
             + b3_ref[...])
        mean = jnp.mean(h, axis=-1, keepdims=True)
        cent = h - mean
        var = jnp.mean(cent * cent, axis=-1, keepdims=True)
        h = cent * lax.rsqrt(var + 1e-5) * g_ref[...] + bt_ref[...]
        out_ref[...] = h + nf

    full = lambda shape: pl.BlockSpec(shape, lambda i: (0,) * len(shape))
    return pl.pallas_call(
        body,
        grid=(grid,),
        in_specs=[
            pl.BlockSpec((NC, BN, D), lambda i: (0, i, 0)),
            pl.BlockSpec((BN, D), lambda i: (i, 0)),
            full((D, D)), full((D, D)), full((1, D)),
            full((D, D)), full((1, D)),
            full((D, D)), full((1, D)),
            full((1, D)), full((1, D)),
        ],
        out_specs=pl.BlockSpec((BN, D), lambda i: (i, 0)),
        out_shape=jax.ShapeDtypeStruct((N_NODES, D), jnp.float32),
    )(partials, node_features, W1a, W1b, b1, W2, b2, W3, b3, gamma, beta)


def kernel(node_features, edge_features, edge_index, W1, b1, W2, b2, W3, b3,
           gamma, beta):
    dst = edge_index[1].astype(jnp.int32)
    chunks = dst.reshape(N_CHUNKS, CH)
    idx_slots = jnp.where(jnp.asarray(_SLOT_VALID)[:, :, None],
                          chunks[jnp.asarray(_SLOT_CHUNK)],
                          jnp.int32(PAD_ROW))

    partials = _sc_segment_sum(edge_features, idx_slots)

    W1a = W1[:D]
    W1b = W1[D:]
    r1 = lambda v: v.reshape(1, D)
    return _tc_mlp(partials, node_features, W1a, W1b, r1(b1), W2, r1(b2),
                   W3, r1(b3), r1(gamma), r1(beta))


# speedup vs baseline: 7.0387x; 1.4254x over previous
import functools

import jax
import jax.numpy as jnp
import numpy as np
from jax import lax
from jax.experimental import pallas as pl
from jax.experimental.pallas import tpu as pltpu
from jax.experimental.pallas import tpu_sc as plsc

N_NODES = 10000
N_EDGES = 320000
D = 128

NC = 2
NS = 16
NW = NC * NS

CH = 128
N_CHUNKS = N_EDGES // CH
SUPER = 1
CPT = 80
NPAIR = CPT // SUPER
ACC_N = 10112
RPT = ACC_N // NS
PAD_ROW = N_NODES

_w = np.arange(NW)
_CNT = (N_CHUNKS // NW) + (_w < (N_CHUNKS % NW)).astype(np.int64)
_START = np.concatenate([[0], np.cumsum(_CNT)[:-1]])
_j = np.arange(CPT)
_SLOT_CHUNK = (_START[:, None]
               + np.minimum(_j[None, :], _CNT[:, None] - 1)).astype(np.int32)
_SLOT_VALID = (_j[None, :] < _CNT[:, None])


def _sc_segment_sum(edge_features, idx_slots):
    mesh = plsc.VectorSubcoreMesh(core_axis_name="c", subcore_axis_name="s")

    @functools.partial(
        pl.kernel,
        out_type=jax.ShapeDtypeStruct((NC, ACC_N, D), jnp.float32),
        mesh=mesh,
        scratch_types=[
            pltpu.VMEM((CPT, CH), jnp.int32),
            pltpu.VMEM((SUPER * CH, D), jnp.float32),
            pltpu.VMEM((SUPER * CH, D), jnp.float32),
            pltpu.VMEM_SHARED((ACC_N, D), jnp.float32),
            pltpu.SemaphoreType.DMA,
            pltpu.SemaphoreType.DMA,
        ],
    )
    def body(edge_hbm, idx_hbm, out_hbm, idx_v, ebufA, ebufB, acc, semA, semB):
        c = lax.axis_index("c")
        s = lax.axis_index("s")
        wid = c * NS + s

        EB = SUPER * CH
        def zrow(r, _):
            for q in range(D // 16):
                ebufA[r, pl.ds(q * 16, 16)] = jnp.zeros((16,), jnp.float32)
            return 0
        lax.fori_loop(0, EB, zrow, 0)
        base_r = s * RPT
        nfull = RPT // EB
        for k in range(nfull):
            pltpu.sync_copy(ebufA, acc.at[pl.ds(base_r + k * EB, EB)])
        rem = RPT - nfull * EB
        if rem:
            pltpu.sync_copy(ebufA.at[pl.ds(0, rem)],
                            acc.at[pl.ds(base_r + nfull * EB, rem)])
        plsc.subcore_barrier()

        pltpu.sync_copy(idx_hbm.at[wid], idx_v)

        nb = jnp.int32(N_CHUNKS // NW)
        rm = jnp.int32(N_CHUNKS % NW)
        start = wid * nb + jnp.minimum(wid, rm)
        max_c0 = jnp.int32(N_CHUNKS - SUPER)

        def load(pair, buf, sem):
            c0 = jnp.minimum(start + pair * SUPER, max_c0)
            return pltpu.async_copy(edge_hbm.at[pl.ds(c0 * CH, EB)], buf, sem)

        def scatter(buf, slot0):
            for q in range(SUPER):
                pltpu.sync_copy(buf.at[pl.ds(q * CH, CH)],
                                acc.at[idx_v.at[slot0 + q]], add=True)

        load(jnp.int32(0), ebufA, semA)
        def pipe(i, _):
            load(2 * i + 1, ebufB, semB)
            pltpu.make_async_copy(edge_hbm.at[pl.ds(0, EB)], ebufA, semA).wait()
            scatter(ebufA, (2 * i) * SUPER)
            load(jnp.minimum(2 * i + 2, jnp.int32(NPAIR - 1)), ebufA, semA)
            pltpu.make_async_copy(edge_hbm.at[pl.ds(0, EB)], ebufB, semB).wait()
            scatter(ebufB, (2 * i + 1) * SUPER)
            return 0
        lax.fori_loop(0, NPAIR // 2, pipe, 0)
        pltpu.make_async_copy(edge_hbm.at[pl.ds(0, EB)], ebufA, semA).wait()

        plsc.subcore_barrier()
        pltpu.sync_copy(acc.at[pl.ds(base_r, RPT)],
                        out_hbm.at[c, pl.ds(base_r, RPT)])

    return body(edge_features, idx_slots)


def _tc_mlp(partials, node_features, W1a, W1b, b1, W2, b2, W3, b3, gamma, beta):
    BN = 1000
    grid = N_NODES // BN

    def body(p_ref, nf_ref, w1a_ref, w1b_ref, b1_ref, w2_ref, b2_ref,
             w3_ref, b3_ref, g_ref, bt_ref, out_ref):
        hd = p_ref[0] + p_ref[1]
        nf = nf_ref[...]
        h = (jnp.dot(hd, w1a_ref[...], preferred_element_type=jnp.float32)
             + jnp.dot(nf, w1b_ref[...], preferred_element_type=jnp.float32)
             + b1_ref[...])
        h = jnp.maximum(h, 0.0)
        h = jnp.maximum(
            jnp.dot(h, w2_ref[...], preferred_element_type=jnp.float32)
            + b2_ref[...], 0.0)
        h = (jnp.dot(h, w3_ref[...], preferred_element_type=jnp.float32)
             + b3_ref[...])
        mean = jnp.mean(h, axis=-1, keepdims=True)
        cent = h - mean
        var = jnp.mean(cent * cent, axis=-1, keepdims=True)
        h = cent * lax.rsqrt(var + 1e-5) * g_ref[...] + bt_ref[...]
        out_ref[...] = h + nf

    full = lambda shape: pl.BlockSpec(shape, lambda i: (0,) * len(shape))
    return pl.pallas_call(
        body,
        grid=(grid,),
        in_specs=[
            pl.BlockSpec((NC, BN, D), lambda i: (0, i, 0)),
            pl.BlockSpec((BN, D), lambda i: (i, 0)),
            full((D, D)), full((D, D)), full((1, D)),
            full((D, D)), full((1, D)),
            full((D, D)), full((1, D)),
            full((1, D)), full((1, D)),
        ],
        out_specs=pl.BlockSpec((BN, D), lambda i: (i, 0)),
        out_shape=jax.ShapeDtypeStruct((N_NODES, D), jnp.float32),
    )(partials, node_features, W1a, W1b, b1, W2, b2, W3, b3, gamma, beta)


def kernel(node_features, edge_features, edge_index, W1, b1, W2, b2, W3, b3,
           gamma, beta):
    dst = edge_index[1].astype(jnp.int32)
    chunks = dst.reshape(N_CHUNKS, CH)
    idx_slots = jnp.where(jnp.asarray(_SLOT_VALID)[:, :, None],
                          chunks[jnp.asarray(_SLOT_CHUNK)],
                          jnp.int32(PAD_ROW))

    partials = _sc_segment_sum(edge_features, idx_slots)

    W1a = W1[:D]
    W1b = W1[D:]
    r1 = lambda v: v.reshape(1, D)
    return _tc_mlp(partials, node_features, W1a, W1b, r1(b1), W2, r1(b2),
                   W3, r1(b3), r1(gamma), r1(beta))


# speedup vs baseline: 7.4297x; 1.0556x over previous
import functools

import jax
import jax.numpy as jnp
from jax import lax
from jax.experimental import pallas as pl
from jax.experimental.pallas import tpu as pltpu
from jax.experimental.pallas import tpu_sc as plsc

N_NODES = 10000
N_EDGES = 320000
D = 128

NC = 2
NS = 16
NW = NC * NS

CH = 128
N_CHUNKS = N_EDGES // CH
SUPER = 1
CPT = 80
NPAIR = CPT // SUPER
ACC_N = 10112
RPT = ACC_N // NS
PAD_ROW = N_NODES
IDX_WIN = CPT + 8
DST_PAD = 2504


def _sc_segment_sum(edge_features, dst_chunks):
    mesh = plsc.VectorSubcoreMesh(core_axis_name="c", subcore_axis_name="s")

    @functools.partial(
        pl.kernel,
        out_type=jax.ShapeDtypeStruct((NC, ACC_N, D), jnp.float32),
        mesh=mesh,
        scratch_types=[
            pltpu.VMEM((IDX_WIN, CH), jnp.int32),
            pltpu.VMEM((SUPER * CH, D), jnp.float32),
            pltpu.VMEM((SUPER * CH, D), jnp.float32),
            pltpu.VMEM_SHARED((ACC_N, D), jnp.float32),
            pltpu.SemaphoreType.DMA,
            pltpu.SemaphoreType.DMA,
        ],
    )
    def body(edge_hbm, idx_hbm, out_hbm, idx_v, ebufA, ebufB, acc, semA, semB):
        c = lax.axis_index("c")
        s = lax.axis_index("s")
        wid = c * NS + s

        EB = SUPER * CH
        def zrow(r, _):
            for q in range(D // 16):
                ebufA[r, pl.ds(q * 16, 16)] = jnp.zeros((16,), jnp.float32)
            return 0
        lax.fori_loop(0, EB, zrow, 0)
        base_r = s * RPT
        nfull = RPT // EB
        for k in range(nfull):
            pltpu.sync_copy(ebufA, acc.at[pl.ds(base_r + k * EB, EB)])
        rem = RPT - nfull * EB
        if rem:
            pltpu.sync_copy(ebufA.at[pl.ds(0, rem)],
                            acc.at[pl.ds(base_r + nfull * EB, rem)])
        plsc.subcore_barrier()

        nb = jnp.int32(N_CHUNKS // NW)
        rm = jnp.int32(N_CHUNKS % NW)
        cnt = nb + jnp.where(wid < rm, jnp.int32(1), jnp.int32(0))
        start = wid * nb + jnp.minimum(wid, rm)
        ws8 = pl.multiple_of(start - lax.rem(start, jnp.int32(8)), 8)
        off = start - ws8

        pltpu.sync_copy(idx_hbm.at[pl.ds(ws8, IDX_WIN)], idx_v)
        def padrow(j, _):
            for q in range(D // 16):
                idx_v[off + j, pl.ds(q * 16, 16)] = jnp.full(
                    (16,), PAD_ROW, jnp.int32)
            return 0
        lax.fori_loop(cnt, CPT, padrow, 0)

        max_c0 = jnp.int32(N_CHUNKS - SUPER)

        def load(pair, buf, sem):
            c0 = jnp.minimum(start + pair * SUPER, max_c0)
            return pltpu.async_copy(edge_hbm.at[pl.ds(c0 * CH, EB)], buf, sem)

        def scatter(buf, slot0):
            for q in range(SUPER):
                pltpu.sync_copy(buf.at[pl.ds(q * CH, CH)],
                                acc.at[idx_v.at[off + slot0 + q]], add=True)

        load(jnp.int32(0), ebufA, semA)
        def pipe(i, _):
            load(2 * i + 1, ebufB, semB)
            pltpu.make_async_copy(edge_hbm.at[pl.ds(0, EB)], ebufA, semA).wait()
            scatter(ebufA, (2 * i) * SUPER)
            load(jnp.minimum(2 * i + 2, jnp.int32(NPAIR - 1)), ebufA, semA)
            pltpu.make_async_copy(edge_hbm.at[pl.ds(0, EB)], ebufB, semB).wait()
            scatter(ebufB, (2 * i + 1) * SUPER)
            return 0
        lax.fori_loop(0, NPAIR // 2, pipe, 0)
        pltpu.make_async_copy(edge_hbm.at[pl.ds(0, EB)], ebufA, semA).wait()

        plsc.subcore_barrier()
        pltpu.sync_copy(acc.at[pl.ds(base_r, RPT)],
                        out_hbm.at[c, pl.ds(base_r, RPT)])

    return body(edge_features, dst_chunks)


def _tc_mlp(partials, node_features, W1a, W1b, b1, W2, b2, W3, b3, gamma, beta):
    BN = 1000
    grid = N_NODES // BN

    def body(p_ref, nf_ref, w1a_ref, w1b_ref, b1_ref, w2_ref, b2_ref,
             w3_ref, b3_ref, g_ref, bt_ref, out_ref):
        hd = p_ref[0] + p_ref[1]
        nf = nf_ref[...]
        h = (jnp.dot(hd, w1a_ref[...], preferred_element_type=jnp.float32)
             + jnp.dot(nf, w1b_ref[...], preferred_element_type=jnp.float32)
             + b1_ref[...])
        h = jnp.maximum(h, 0.0)
        h = jnp.maximum(
            jnp.dot(h, w2_ref[...], preferred_element_type=jnp.float32)
            + b2_ref[...], 0.0)
        h = (jnp.dot(h, w3_ref[...], preferred_element_type=jnp.float32)
             + b3_ref[...])
        mean = jnp.mean(h, axis=-1, keepdims=True)
        cent = h - mean
        var = jnp.mean(cent * cent, axis=-1, keepdims=True)
        h = cent * lax.rsqrt(var + 1e-5) * g_ref[...] + bt_ref[...]
        out_ref[...] = h + nf

    full = lambda shape: pl.BlockSpec(shape, lambda i: (0,) * len(shape))
    return pl.pallas_call(
        body,
        grid=(grid,),
        in_specs=[
            pl.BlockSpec((NC, BN, D), lambda i: (0, i, 0)),
            pl.BlockSpec((BN, D), lambda i: (i, 0)),
            full((D, D)), full((D, D)), full((1, D)),
            full((D, D)), full((1, D)),
            full((D, D)), full((1, D)),
            full((1, D)), full((1, D)),
        ],
        out_specs=pl.BlockSpec((BN, D), lambda i: (i, 0)),
        out_shape=jax.ShapeDtypeStruct((N_NODES, D), jnp.float32),
    )(partials, node_features, W1a, W1b, b1, W2, b2, W3, b3, gamma, beta)


def kernel(node_features, edge_features, edge_index, W1, b1, W2, b2, W3, b3,
           gamma, beta):
    dst = edge_index[1].astype(jnp.int32).reshape(N_CHUNKS, CH)
    dst_chunks = jnp.concatenate(
        [dst, jnp.full((DST_PAD - N_CHUNKS, CH), PAD_ROW, jnp.int32)], axis=0)

    partials = _sc_segment_sum(edge_features, dst_chunks)

    W1a = W1[:D]
    W1b = W1[D:]
    r1 = lambda v: v.reshape(1, D)
    return _tc_mlp(partials, node_features, W1a, W1b, r1(b1), W2, r1(b2),
                   W3, r1(b3), r1(gamma), r1(beta))


# speedup vs baseline: 7.7225x; 1.0394x over previous
import functools

import jax
import jax.numpy as jnp
from jax import lax
from jax.experimental import pallas as pl
from jax.experimental.pallas import tpu as pltpu
from jax.experimental.pallas import tpu_sc as plsc

N_NODES = 10000
N_EDGES = 320000
D = 128

NC = 2
NS = 16
NW = NC * NS

CH = 128
N_CHUNKS = N_EDGES // CH
NB = 78
ACC_N = 10112
RPT = ACC_N // NS
IDX_WIN = 88
WS_MAX = N_CHUNKS - IDX_WIN


def _sc_segment_sum(edge_features, dst_chunks):
    mesh = plsc.VectorSubcoreMesh(core_axis_name="c", subcore_axis_name="s")

    @functools.partial(
        pl.kernel,
        out_type=jax.ShapeDtypeStruct((NC, ACC_N, D), jnp.float32),
        mesh=mesh,
        scratch_types=[
            pltpu.VMEM((IDX_WIN, CH), jnp.int32),
            pltpu.VMEM((CH, D), jnp.float32),
            pltpu.VMEM((CH, D), jnp.float32),
            pltpu.VMEM_SHARED((ACC_N, D), jnp.float32),
            pltpu.SemaphoreType.DMA,
            pltpu.SemaphoreType.DMA,
        ],
    )
    def body(edge_hbm, idx_hbm, out_hbm, idx_v, ebufA, ebufB, acc, semA, semB):
        c = lax.axis_index("c")
        s = lax.axis_index("s")
        wid = c * NS + s

        EB = CH
        def zrow(r, _):
            for q in range(D // 16):
                ebufA[r, pl.ds(q * 16, 16)] = jnp.zeros((16,), jnp.float32)
            return 0
        lax.fori_loop(0, EB, zrow, 0)
        base_r = s * RPT
        nfull = RPT // EB
        for k in range(nfull):
            pltpu.sync_copy(ebufA, acc.at[pl.ds(base_r + k * EB, EB)])
        rem = RPT - nfull * EB
        if rem:
            pltpu.sync_copy(ebufA.at[pl.ds(0, rem)],
                            acc.at[pl.ds(base_r + nfull * EB, rem)])
        plsc.subcore_barrier()

        cnt = jnp.where(wid < 2, jnp.int32(79),
                        jnp.where(wid < 31, jnp.int32(78), jnp.int32(80)))
        start = jnp.where(wid < 2, wid * jnp.int32(79),
                          wid * jnp.int32(78) + jnp.int32(2))
        ws8 = pl.multiple_of(
            jnp.minimum(start - lax.rem(start, jnp.int32(8)),
                        jnp.int32(WS_MAX)), 8)
        off = start - ws8

        pltpu.sync_copy(idx_hbm.at[pl.ds(ws8, IDX_WIN)], idx_v)

        def load(slot, buf, sem):
            return pltpu.async_copy(
                edge_hbm.at[pl.ds((start + slot) * CH, CH)], buf, sem)

        def scatter(buf, slot):
            pltpu.sync_copy(buf, acc.at[idx_v.at[off + slot]], add=True)

        def wait(buf, sem):
            pltpu.make_async_copy(edge_hbm.at[pl.ds(0, CH)], buf, sem).wait()

        load(jnp.int32(0), ebufA, semA)
        def pipe(i, _):
            load(2 * i + 1, ebufB, semB)
            wait(ebufA, semA)
            scatter(ebufA, 2 * i)
            load(jnp.minimum(2 * i + 2, jnp.int32(NB - 1)), ebufA, semA)
            wait(ebufB, semB)
            scatter(ebufB, 2 * i + 1)
            return 0
        lax.fori_loop(0, NB // 2, pipe, 0)
        wait(ebufA, semA)

        @pl.when(cnt > NB)
        def _():
            pltpu.sync_copy(edge_hbm.at[pl.ds((start + NB) * CH, CH)], ebufA)
            scatter(ebufA, jnp.int32(NB))
        @pl.when(cnt > NB + 1)
        def _():
            pltpu.sync_copy(edge_hbm.at[pl.ds((start + NB + 1) * CH, CH)],
                            ebufA)
            scatter(ebufA, jnp.int32(NB + 1))

        plsc.subcore_barrier()
        pltpu.sync_copy(acc.at[pl.ds(base_r, RPT)],
                        out_hbm.at[c, pl.ds(base_r, RPT)])

    return body(edge_features, dst_chunks)


def _tc_mlp(partials, node_features, W1a, W1b, b1, W2, b2, W3, b3, gamma, beta):
    BN = 2000
    grid = N_NODES // BN

    def body(p_ref, nf_ref, w1a_ref, w1b_ref, b1_ref, w2_ref, b2_ref,
             w3_ref, b3_ref, g_ref, bt_ref, out_ref):
        hd = p_ref[0] + p_ref[1]
        nf = nf_ref[...]
        h = (jnp.dot(hd, w1a_ref[...], preferred_element_type=jnp.float32)
             + jnp.dot(nf, w1b_ref[...], preferred_element_type=jnp.float32)
             + b1_ref[...])
        h = jnp.maximum(h, 0.0)
        h = jnp.maximum(
            jnp.dot(h, w2_ref[...], preferred_element_type=jnp.float32)
            + b2_ref[...], 0.0)
        h = (jnp.dot(h, w3_ref[...], preferred_element_type=jnp.float32)
             + b3_ref[...])
        mean = jnp.mean(h, axis=-1, keepdims=True)
        cent = h - mean
        var = jnp.mean(cent * cent, axis=-1, keepdims=True)
        h = cent * lax.rsqrt(var + 1e-5) * g_ref[...] + bt_ref[...]
        out_ref[...] = h + nf

    full = lambda shape: pl.BlockSpec(shape, lambda i: (0,) * len(shape))
    return pl.pallas_call(
        body,
        grid=(grid,),
        in_specs=[
            pl.BlockSpec((NC, BN, D), lambda i: (0, i, 0)),
            pl.BlockSpec((BN, D), lambda i: (i, 0)),
            full((D, D)), full((D, D)), full((1, D)),
            full((D, D)), full((1, D)),
            full((D, D)), full((1, D)),
            full((1, D)), full((1, D)),
        ],
        out_specs=pl.BlockSpec((BN, D), lambda i: (i, 0)),
        out_shape=jax.ShapeDtypeStruct((N_NODES, D), jnp.float32),
    )(partials, node_features, W1a, W1b, b1, W2, b2, W3, b3, gamma, beta)


def kernel(node_features, edge_features, edge_index, W1, b1, W2, b2, W3, b3,
           gamma, beta):
    dst_chunks = edge_index[1].astype(jnp.int32).reshape(N_CHUNKS, CH)
    partials = _sc_segment_sum(edge_features, dst_chunks)

    W1a = W1[:D]
    W1b = W1[D:]
    r1 = lambda v: v.reshape(1, D)
    return _tc_mlp(partials, node_features, W1a, W1b, r1(b1), W2, r1(b2),
                   W3, r1(b3), r1(gamma), r1(beta))
